# baseline (speedup 1.0000x reference)
import functools

import jax
import jax.numpy as jnp
from jax import lax
from jax.experimental import pallas as pl
from jax.experimental.pallas import tpu as pltpu
from jax.experimental.pallas import tpu_sc as plsc


def _make_gather(NB, V, D, NC, NS):
    NW = NC * NS
    BPW = NB // NW
    C = 32
    NBUF = 2
    NCH = BPW // C

    mesh = plsc.VectorSubcoreMesh(core_axis_name="c", subcore_axis_name="s")

    @functools.partial(
        pl.kernel,
        mesh=mesh,
        out_type=jax.ShapeDtypeStruct((NB, D), jnp.float32),
        scratch_types=[
            pltpu.VMEM((BPW,), jnp.int32),
            [pltpu.VMEM((C, D), jnp.float32)] * NBUF,
            [pltpu.SemaphoreType.DMA] * NBUF,
            [pltpu.SemaphoreType.DMA] * NBUF,
            pltpu.VMEM_SHARED((V, D), jnp.float32),
        ],
        compiler_params=pltpu.CompilerParams(use_tc_tiling_on_sc=False),
    )
    def gather_kernel(
        idx_hbm, table_hbm, out_hbm, idx_v, rows, gsems, osems, table_sh
    ):
        wid = lax.axis_index("s") * NC + lax.axis_index("c")
        base = wid * BPW

        @pl.when(lax.axis_index("s") == 0)
        def _stage():
            pltpu.sync_copy(table_hbm, table_sh)

        pltpu.sync_copy(idx_hbm.at[pl.ds(base, BPW)], idx_v)
        plsc.subcore_barrier()

        def gather_desc(j, b):
            return pltpu.make_async_copy(
                table_sh.at[idx_v.at[pl.ds(j * C, C)]], rows[b], gsems[b]
            )

        def out_desc(j, b):
            return pltpu.make_async_copy(
                rows[b], out_hbm.at[pl.ds(base + j * C, C)], osems[b]
            )

        for b in range(NBUF):
            gather_desc(b, b).start()

        def body(p, _):
            j0 = p * NBUF
            for b in range(NBUF):
                gather_desc(j0 + b, b).wait()
                out_desc(j0 + b, b).start()
            for b in range(NBUF):
                out_desc(j0 + b, b).wait()
                gather_desc(j0 + NBUF + b, b).start()
            return 0

        lax.fori_loop(0, NCH // NBUF - 1, body, 0)

        j0 = NCH - NBUF
        for b in range(NBUF):
            gather_desc(j0 + b, b).wait()
            out_desc(j0 + b, b).start()
        for b in range(NBUF):
            out_desc(j0 + b, b).wait()

    return gather_kernel


def kernel(indices, table):
    B, H = indices.shape
    V, D = table.shape
    NSLAB = 4
    BS = B // NSLAB
    flat_idx = indices.reshape(B * H).astype(jnp.int32)
    info = plsc.get_sparse_core_info()
    gather = _make_gather(BS * H, V, D, info.num_cores, info.num_subcores)
    slabs = []
    for s in range(NSLAB):
        flat = gather(
            lax.slice_in_dim(flat_idx, s * BS * H, (s + 1) * BS * H), table
        )
        slabs.append(flat.reshape(BS, H, D))
    return jnp.concatenate(slabs, axis=0)


# speedup vs baseline: 1.5161x; 1.5161x over previous
import functools

import jax
import jax.numpy as jnp
from jax import lax
from jax.experimental import pallas as pl
from jax.experimental.pallas import tpu as pltpu
from jax.experimental.pallas import tpu_sc as plsc


def _make_gather(NB, V, D, NC, NS):
    NW = NC * NS
    BPW = NB // NW
    C = 16
    NBUF = 4
    NCH = BPW // C

    mesh = plsc.VectorSubcoreMesh(core_axis_name="c", subcore_axis_name="s")

    @functools.partial(
        pl.kernel,
        mesh=mesh,
        out_type=jax.ShapeDtypeStruct((NB, D), jnp.float32),
        scratch_types=[
            pltpu.VMEM((BPW,), jnp.int32),
            [pltpu.VMEM((C, D), jnp.float32)] * NBUF,
            [pltpu.SemaphoreType.DMA] * NBUF,
            [pltpu.SemaphoreType.DMA] * NBUF,
            pltpu.VMEM_SHARED((V, D), jnp.float32),
        ],
        compiler_params=pltpu.CompilerParams(use_tc_tiling_on_sc=False),
    )
    def gather_kernel(
        idx_hbm, table_hbm, out_hbm, idx_v, rows, gsems, osems, table_sh
    ):
        wid = lax.axis_index("s") * NC + lax.axis_index("c")
        base = wid * BPW

        @pl.when(lax.axis_index("s") == 0)
        def _stage():
            pltpu.sync_copy(table_hbm, table_sh)

        pltpu.sync_copy(idx_hbm.at[pl.ds(base, BPW)], idx_v)
        plsc.subcore_barrier()

        def gather_desc(j, b):
            return pltpu.make_async_copy(
                table_sh.at[idx_v.at[pl.ds(j * C, C)]], rows[b], gsems[b]
            )

        def out_desc(j, b):
            return pltpu.make_async_copy(
                rows[b], out_hbm.at[pl.ds(base + j * C, C)], osems[b]
            )

        for b in range(NBUF):
            gather_desc(b, b).start()

        def body(p, _):
            j0 = p * NBUF
            for b in range(NBUF):
                gather_desc(j0 + b, b).wait()
                out_desc(j0 + b, b).start()
            for b in range(NBUF):
                out_desc(j0 + b, b).wait()
                gather_desc(j0 + NBUF + b, b).start()
            return 0

        lax.fori_loop(0, NCH // NBUF - 1, body, 0)

        j0 = NCH - NBUF
        for b in range(NBUF):
            gather_desc(j0 + b, b).wait()
            out_desc(j0 + b, b).start()
        for b in range(NBUF):
            out_desc(j0 + b, b).wait()

    return gather_kernel


def kernel(indices, table):
    B, H = indices.shape
    V, D = table.shape
    flat_idx = indices.reshape(B * H).astype(jnp.int32)
    info = plsc.get_sparse_core_info()
    out = _make_gather(B * H, V, D, info.num_cores, info.num_subcores)(
        flat_idx, table
    )
    return out.reshape(B, H, D)


# speedup vs baseline: 1.5174x; 1.0009x over previous
import functools

import jax
import jax.numpy as jnp
from jax import lax
from jax.experimental import pallas as pl
from jax.experimental.pallas import tpu as pltpu
from jax.experimental.pallas import tpu_sc as plsc


def _make_gather(NB, V, D, NC, NS):
    NW = NC * NS
    BPW = NB // NW
    C = 8
    NBUF = 8
    NCH = BPW // C

    mesh = plsc.VectorSubcoreMesh(core_axis_name="c", subcore_axis_name="s")

    @functools.partial(
        pl.kernel,
        mesh=mesh,
        out_type=jax.ShapeDtypeStruct((NB, D), jnp.float32),
        scratch_types=[
            pltpu.VMEM((BPW,), jnp.int32),
            [pltpu.VMEM((C, D), jnp.float32)] * NBUF,
            [pltpu.SemaphoreType.DMA] * NBUF,
            [pltpu.SemaphoreType.DMA] * NBUF,
            pltpu.VMEM_SHARED((V, D), jnp.float32),
        ],
        compiler_params=pltpu.CompilerParams(use_tc_tiling_on_sc=False),
    )
    def gather_kernel(
        idx_hbm, table_hbm, out_hbm, idx_v, rows, gsems, osems, table_sh
    ):
        wid = lax.axis_index("s") * NC + lax.axis_index("c")
        base = wid * BPW

        @pl.when(lax.axis_index("s") == 0)
        def _stage():
            pltpu.sync_copy(table_hbm, table_sh)

        pltpu.sync_copy(idx_hbm.at[pl.ds(base, BPW)], idx_v)
        plsc.subcore_barrier()

        def gather_desc(j, b):
            return pltpu.make_async_copy(
                table_sh.at[idx_v.at[pl.ds(j * C, C)]], rows[b], gsems[b]
            )

        def out_desc(j, b):
            return pltpu.make_async_copy(
                rows[b], out_hbm.at[pl.ds(base + j * C, C)], osems[b]
            )

        for b in range(NBUF):
            gather_desc(b, b).start()

        def body(p, _):
            j0 = p * NBUF
            for b in range(NBUF):
                gather_desc(j0 + b, b).wait()
                out_desc(j0 + b, b).start()
            for b in range(NBUF):
                out_desc(j0 + b, b).wait()
                gather_desc(j0 + NBUF + b, b).start()
            return 0

        lax.fori_loop(0, NCH // NBUF - 1, body, 0)

        j0 = NCH - NBUF
        for b in range(NBUF):
            gather_desc(j0 + b, b).wait()
            out_desc(j0 + b, b).start()
        for b in range(NBUF):
            out_desc(j0 + b, b).wait()

    return gather_kernel


def kernel(indices, table):
    B, H = indices.shape
    V, D = table.shape
    flat_idx = indices.reshape(B * H).astype(jnp.int32)
    info = plsc.get_sparse_core_info()
    out = _make_gather(B * H, V, D, info.num_cores, info.num_subcores)(
        flat_idx, table
    )
    return out.reshape(B, H, D)
